# baseline (speedup 1.0000x reference)
import functools

import jax
import jax.numpy as jnp
from jax import lax
from jax.experimental import pallas as pl
from jax.experimental.pallas import tpu as pltpu
from jax.experimental.pallas import tpu_sc as plsc


ROWS = 512


def _add_kernel(x_ref, pos_ref, out_ref):
    out_ref[...] = x_ref[...] + pos_ref[...][:, None, :]


def _kernel_tc(x, pos_table):
    chunk, b, d = x.shape
    grid = (chunk // ROWS,)
    return pl.pallas_call(
        _add_kernel,
        grid=grid,
        in_specs=[
            pl.BlockSpec((ROWS, b, d), lambda i: (i, 0, 0)),
            pl.BlockSpec((ROWS, d), lambda i: (i, 0)),
        ],
        out_specs=pl.BlockSpec((ROWS, b, d), lambda i: (i, 0, 0)),
        out_shape=jax.ShapeDtypeStruct((chunk, b, d), x.dtype),
    )(x, pos_table[:chunk])


NW = 32
G = 8
LANES = 16


def _kernel_sc(x, pos_table):
    chunk, b, d = x.shape
    per_w = chunk // NW
    steps = per_w // G
    xf = x.reshape(chunk * b * d)
    pf = pos_table[:chunk].reshape(chunk * d)

    mesh = plsc.VectorSubcoreMesh(core_axis_name="c", subcore_axis_name="s")

    @functools.partial(
        pl.kernel,
        mesh=mesh,
        out_type=jax.ShapeDtypeStruct((chunk * b * d,), jnp.float32),
        scratch_types=[
            pltpu.VMEM((G * d,), jnp.float32),
            pltpu.VMEM((G * b * d,), jnp.float32),
        ],
    )
    def k(x_hbm, pos_hbm, out_hbm, pos_v, x_v):
        wid = lax.axis_index("s") * 2 + lax.axis_index("c")
        i_base = wid * per_w

        def step_body(s, carry):
            i0 = i_base + s * G
            pltpu.sync_copy(pos_hbm.at[pl.ds(i0 * d, G * d)], pos_v)
            pltpu.sync_copy(x_hbm.at[pl.ds(i0 * b * d, G * b * d)], x_v)

            def add_body(t, c):
                g = t // (d // LANES)
                j = t % (d // LANES)
                pv = pos_v[pl.ds(g * d + j * LANES, LANES)]
                xbase = g * b * d + j * LANES
                for bb in range(b):
                    off = xbase + bb * d
                    x_v[pl.ds(off, LANES)] = x_v[pl.ds(off, LANES)] + pv
                return c

            lax.fori_loop(0, G * (d // LANES), add_body, 0)
            pltpu.sync_copy(x_v, out_hbm.at[pl.ds(i0 * b * d, G * b * d)])
            return carry

        lax.fori_loop(0, steps, step_body, 0)

    return k(xf, pf).reshape(chunk, b, d)


def kernel(x, pos_table):
    return _kernel_sc(x, pos_table)


# speedup vs baseline: 1.3162x; 1.3162x over previous
import functools

import jax
import jax.numpy as jnp
from jax import lax
from jax.experimental import pallas as pl
from jax.experimental.pallas import tpu as pltpu
from jax.experimental.pallas import tpu_sc as plsc


ROWS = 512


def _add_kernel(x_ref, pos_ref, out_ref):
    out_ref[...] = x_ref[...] + pos_ref[...][:, None, :]


def _kernel_tc(x, pos_table):
    chunk, b, d = x.shape
    grid = (chunk // ROWS,)
    return pl.pallas_call(
        _add_kernel,
        grid=grid,
        in_specs=[
            pl.BlockSpec((ROWS, b, d), lambda i: (i, 0, 0)),
            pl.BlockSpec((ROWS, d), lambda i: (i, 0)),
        ],
        out_specs=pl.BlockSpec((ROWS, b, d), lambda i: (i, 0, 0)),
        out_shape=jax.ShapeDtypeStruct((chunk, b, d), x.dtype),
    )(x, pos_table[:chunk])


NW = 32
G = 4
NBUF = 4
LANES = 16
UNROLL = 4


def _kernel_sc(x, pos_table):
    chunk, b, d = x.shape
    per_w = chunk // NW
    steps = per_w // G
    xf = x.reshape(chunk * b * d)
    pf = pos_table[:chunk].reshape(chunk * d)

    mesh = plsc.VectorSubcoreMesh(core_axis_name="c", subcore_axis_name="s")

    scratch = (
        [pltpu.VMEM((G * d,), jnp.float32) for _ in range(NBUF)]
        + [pltpu.VMEM((G * b * d,), jnp.float32) for _ in range(NBUF)]
        + [pltpu.SemaphoreType.DMA for _ in range(2 * NBUF)]
    )

    @functools.partial(
        pl.kernel,
        mesh=mesh,
        out_type=jax.ShapeDtypeStruct((chunk * b * d,), jnp.float32),
        scratch_types=scratch,
    )
    def k(x_hbm, pos_hbm, out_hbm, *bufs):
        pos_v = bufs[0:NBUF]
        x_v = bufs[NBUF:2 * NBUF]
        in_sem = bufs[2 * NBUF:3 * NBUF]
        out_sem = bufs[3 * NBUF:4 * NBUF]

        wid = lax.axis_index("s") * 2 + lax.axis_index("c")
        i_base = wid * per_w

        def issue_in(s, p):
            i0 = i_base + s * G
            pltpu.async_copy(pos_hbm.at[pl.ds(i0 * d, G * d)], pos_v[p],
                             in_sem[p])
            pltpu.async_copy(x_hbm.at[pl.ds(i0 * b * d, G * b * d)], x_v[p],
                             in_sem[p])

        def wait_in(p):
            pltpu.make_async_copy(pos_hbm.at[pl.ds(0, G * d)], pos_v[p],
                                  in_sem[p]).wait()
            pltpu.make_async_copy(x_hbm.at[pl.ds(0, G * b * d)], x_v[p],
                                  in_sem[p]).wait()

        def issue_out(s, p):
            i0 = i_base + s * G
            pltpu.async_copy(x_v[p], out_hbm.at[pl.ds(i0 * b * d, G * b * d)],
                             out_sem[p])

        def wait_out(p):
            pltpu.make_async_copy(x_v[p], out_hbm.at[pl.ds(0, G * b * d)],
                                  out_sem[p]).wait()

        def compute(p):
            pv_ref = pos_v[p]
            xv_ref = x_v[p]

            def body(t, c):
                g = t >> 4
                j4 = t & 15
                for u in range(UNROLL):
                    coff = (j4 * UNROLL + u) * LANES
                    pv = pv_ref[pl.ds(g * d + coff, LANES)]
                    xb = g * b * d + coff
                    for bb in range(b):
                        off = xb + bb * d
                        xv_ref[pl.ds(off, LANES)] = (
                            xv_ref[pl.ds(off, LANES)] + pv)
                return c

            lax.fori_loop(0, G * d // (LANES * UNROLL), body, 0)

        issue_in(0, 0)
        issue_in(1, 1)

        for s in (0, 1):
            p = s % NBUF
            wait_in(p)
            compute(p)
            issue_out(s, p)
            issue_in(s + 2, (s + 2) % NBUF)

        def steady(it, carry):
            for p0 in range(NBUF):
                s = 2 + it * NBUF + p0
                p = (2 + p0) % NBUF
                wait_in(p)
                compute(p)
                issue_out(s, p)
                r = (p + 2) % NBUF
                wait_out(r)
                issue_in(s + 2, r)
            return carry

        lax.fori_loop(0, (steps - 4) // NBUF, steady, 0)

        for s in (steps - 2, steps - 1):
            p = s % NBUF
            wait_in(p)
            compute(p)
            issue_out(s, p)

        for p in range(NBUF):
            wait_out(p)

    return k(xf, pf).reshape(chunk, b, d)


def kernel(x, pos_table):
    return _kernel_sc(x, pos_table)


# speedup vs baseline: 3.8302x; 2.9099x over previous
import functools

import jax
import jax.numpy as jnp
from jax import lax
from jax.experimental import pallas as pl
from jax.experimental.pallas import tpu as pltpu
from jax.experimental.pallas import tpu_sc as plsc


ROWS = 512


def _add_kernel(x_ref, pos_ref, out_ref):
    out_ref[...] = x_ref[...] + pos_ref[...][:, None, :]


def _kernel_tc(x, pos_table):
    chunk, b, d = x.shape
    grid = (chunk // ROWS,)
    return pl.pallas_call(
        _add_kernel,
        grid=grid,
        in_specs=[
            pl.BlockSpec((ROWS, b, d), lambda i: (i, 0, 0)),
            pl.BlockSpec((ROWS, d), lambda i: (i, 0)),
        ],
        out_specs=pl.BlockSpec((ROWS, b, d), lambda i: (i, 0, 0)),
        out_shape=jax.ShapeDtypeStruct((chunk, b, d), x.dtype),
    )(x, pos_table[:chunk])


NW = 32
G = 4
NBUF = 4
LANES = 16
UNROLL = 4


def _kernel_sc(x, pos_table):
    chunk, b, d = x.shape
    per_w = chunk // NW
    steps = per_w // G

    mesh = plsc.VectorSubcoreMesh(core_axis_name="c", subcore_axis_name="s")

    scratch = (
        [pltpu.VMEM((G, d), jnp.float32) for _ in range(NBUF)]
        + [pltpu.VMEM((G, b, d), jnp.float32) for _ in range(NBUF)]
        + [pltpu.SemaphoreType.DMA for _ in range(2 * NBUF)]
    )

    @functools.partial(
        pl.kernel,
        mesh=mesh,
        out_type=jax.ShapeDtypeStruct((chunk, b, d), jnp.float32),
        scratch_types=scratch,
    )
    def k(x_hbm, pos_hbm, out_hbm, *bufs):
        pos_v = bufs[0:NBUF]
        x_v = bufs[NBUF:2 * NBUF]
        in_sem = bufs[2 * NBUF:3 * NBUF]
        out_sem = bufs[3 * NBUF:4 * NBUF]

        wid = lax.axis_index("s") * 2 + lax.axis_index("c")
        i_base = wid * per_w

        def issue_in(s, p):
            i0 = i_base + s * G
            pltpu.async_copy(pos_hbm.at[pl.ds(i0, G)], pos_v[p], in_sem[p])
            pltpu.async_copy(x_hbm.at[pl.ds(i0, G)], x_v[p], in_sem[p])

        def wait_in(p):
            pltpu.make_async_copy(pos_hbm.at[pl.ds(0, G)], pos_v[p],
                                  in_sem[p]).wait()
            pltpu.make_async_copy(x_hbm.at[pl.ds(0, G)], x_v[p],
                                  in_sem[p]).wait()

        def issue_out(s, p):
            i0 = i_base + s * G
            pltpu.async_copy(x_v[p], out_hbm.at[pl.ds(i0, G)], out_sem[p])

        def wait_out(p):
            pltpu.make_async_copy(x_v[p], out_hbm.at[pl.ds(0, G)],
                                  out_sem[p]).wait()

        def compute(p):
            pv_ref = pos_v[p]
            xv_ref = x_v[p]

            def body(t, c):
                g = t >> 4
                j4 = t & 15
                for u in range(UNROLL):
                    coff = (j4 * UNROLL + u) * LANES
                    pv = pv_ref[g, pl.ds(coff, LANES)]
                    for bb in range(b):
                        xv_ref[g, bb, pl.ds(coff, LANES)] = (
                            xv_ref[g, bb, pl.ds(coff, LANES)] + pv)
                return c

            lax.fori_loop(0, G * d // (LANES * UNROLL), body, 0)

        issue_in(0, 0)
        issue_in(1, 1)

        for s in (0, 1):
            p = s % NBUF
            wait_in(p)
            compute(p)
            issue_out(s, p)
            issue_in(s + 2, (s + 2) % NBUF)

        def steady(it, carry):
            for p0 in range(NBUF):
                s = 2 + it * NBUF + p0
                p = (2 + p0) % NBUF
                wait_in(p)
                compute(p)
                issue_out(s, p)
                r = (p + 2) % NBUF
                wait_out(r)
                issue_in(s + 2, r)
            return carry

        lax.fori_loop(0, (steps - 4) // NBUF, steady, 0)

        for s in (steps - 2, steps - 1):
            p = s % NBUF
            wait_in(p)
            compute(p)
            issue_out(s, p)

        for p in range(NBUF):
            wait_out(p)

    return k(x, pos_table[:chunk])


def kernel(x, pos_table):
    return _kernel_sc(x, pos_table)


# speedup vs baseline: 4.0543x; 1.0585x over previous
import functools

import jax
import jax.numpy as jnp
from jax import lax
from jax.experimental import pallas as pl
from jax.experimental.pallas import tpu as pltpu
from jax.experimental.pallas import tpu_sc as plsc


ROWS = 512


def _add_kernel(x_ref, pos_ref, out_ref):
    out_ref[...] = x_ref[...] + pos_ref[...][:, None, :]


def _kernel_tc(x, pos_table):
    chunk, b, d = x.shape
    grid = (chunk // ROWS,)
    return pl.pallas_call(
        _add_kernel,
        grid=grid,
        in_specs=[
            pl.BlockSpec((ROWS, b, d), lambda i: (i, 0, 0)),
            pl.BlockSpec((ROWS, d), lambda i: (i, 0)),
        ],
        out_specs=pl.BlockSpec((ROWS, b, d), lambda i: (i, 0, 0)),
        out_shape=jax.ShapeDtypeStruct((chunk, b, d), x.dtype),
    )(x, pos_table[:chunk])


NW = 32
G = 8
NBUF = 3
LANES = 16
UNROLL = 4


def _kernel_sc(x, pos_table):
    chunk, b, d = x.shape
    per_w = chunk // NW
    steps = per_w // G

    mesh = plsc.VectorSubcoreMesh(core_axis_name="c", subcore_axis_name="s")

    scratch = (
        [pltpu.VMEM((G, d), jnp.float32) for _ in range(NBUF)]
        + [pltpu.VMEM((G, b, d), jnp.float32) for _ in range(NBUF)]
        + [pltpu.SemaphoreType.DMA for _ in range(2 * NBUF)]
    )

    @functools.partial(
        pl.kernel,
        mesh=mesh,
        out_type=jax.ShapeDtypeStruct((chunk, b, d), jnp.float32),
        scratch_types=scratch,
    )
    def k(x_hbm, pos_hbm, out_hbm, *bufs):
        pos_v = bufs[0:NBUF]
        x_v = bufs[NBUF:2 * NBUF]
        in_sem = bufs[2 * NBUF:3 * NBUF]
        out_sem = bufs[3 * NBUF:4 * NBUF]

        wid = lax.axis_index("s") * 2 + lax.axis_index("c")
        i_base = wid * per_w

        def issue_in(s, p):
            i0 = i_base + s * G
            pltpu.async_copy(pos_hbm.at[pl.ds(i0, G)], pos_v[p], in_sem[p])
            pltpu.async_copy(x_hbm.at[pl.ds(i0, G)], x_v[p], in_sem[p])

        def wait_in(p):
            pltpu.make_async_copy(pos_hbm.at[pl.ds(0, G)], pos_v[p],
                                  in_sem[p]).wait()
            pltpu.make_async_copy(x_hbm.at[pl.ds(0, G)], x_v[p],
                                  in_sem[p]).wait()

        def issue_out(s, p):
            i0 = i_base + s * G
            pltpu.async_copy(x_v[p], out_hbm.at[pl.ds(i0, G)], out_sem[p])

        def wait_out(p):
            pltpu.make_async_copy(x_v[p], out_hbm.at[pl.ds(0, G)],
                                  out_sem[p]).wait()

        def compute(p):
            pv_ref = pos_v[p]
            xv_ref = x_v[p]

            def body(t, c):
                g = t >> 4
                j4 = t & 15
                for u in range(UNROLL):
                    coff = (j4 * UNROLL + u) * LANES
                    pv = pv_ref[g, pl.ds(coff, LANES)]
                    for bb in range(b):
                        xv_ref[g, bb, pl.ds(coff, LANES)] = (
                            xv_ref[g, bb, pl.ds(coff, LANES)] + pv)
                return c

            lax.fori_loop(0, G * d // (LANES * UNROLL), body, 0)

        issue_in(0, 0)
        issue_in(1, 1)

        for s in (0, 1):
            p = s % NBUF
            wait_in(p)
            compute(p)
            issue_out(s, p)
            r = (s + 2) % NBUF
            if s + 2 - NBUF >= 0:
                wait_out(r)
            issue_in(s + 2, r)

        n_steady = ((steps - 4) // NBUF) * NBUF

        def steady(it, carry):
            for p0 in range(NBUF):
                s = 2 + it * NBUF + p0
                p = (2 + p0) % NBUF
                wait_in(p)
                compute(p)
                issue_out(s, p)
                r = (p + 2) % NBUF
                wait_out(r)
                issue_in(s + 2, r)
            return carry

        lax.fori_loop(0, n_steady // NBUF, steady, 0)

        for s in range(2 + n_steady, steps):
            p = s % NBUF
            wait_in(p)
            compute(p)
            issue_out(s, p)
            if s + 2 < steps:
                r = (p + 2) % NBUF
                wait_out(r)
                issue_in(s + 2, r)

        for p in range(NBUF):
            wait_out(p)

    return k(x, pos_table[:chunk])


def kernel(x, pos_table):
    return _kernel_sc(x, pos_table)


# speedup vs baseline: 4.1211x; 1.0165x over previous
import functools

import jax
import jax.numpy as jnp
from jax import lax
from jax.experimental import pallas as pl
from jax.experimental.pallas import tpu as pltpu
from jax.experimental.pallas import tpu_sc as plsc


ROWS = 512


def _add_kernel(x_ref, pos_ref, out_ref):
    out_ref[...] = x_ref[...] + pos_ref[...][:, None, :]


def _kernel_tc(x, pos_table):
    chunk, b, d = x.shape
    grid = (chunk // ROWS,)
    return pl.pallas_call(
        _add_kernel,
        grid=grid,
        in_specs=[
            pl.BlockSpec((ROWS, b, d), lambda i: (i, 0, 0)),
            pl.BlockSpec((ROWS, d), lambda i: (i, 0)),
        ],
        out_specs=pl.BlockSpec((ROWS, b, d), lambda i: (i, 0, 0)),
        out_shape=jax.ShapeDtypeStruct((chunk, b, d), x.dtype),
    )(x, pos_table[:chunk])


NW = 32
G = 4
NBUF = 6
LEAD = 3
LANES = 16
UNROLL = 4


def _kernel_sc(x, pos_table):
    chunk, b, d = x.shape
    per_w = chunk // NW
    steps = per_w // G

    mesh = plsc.VectorSubcoreMesh(core_axis_name="c", subcore_axis_name="s")

    scratch = (
        [pltpu.VMEM((G, d), jnp.float32) for _ in range(NBUF)]
        + [pltpu.VMEM((G, b, d), jnp.float32) for _ in range(NBUF)]
        + [pltpu.SemaphoreType.DMA for _ in range(2 * NBUF)]
    )

    @functools.partial(
        pl.kernel,
        mesh=mesh,
        out_type=jax.ShapeDtypeStruct((chunk, b, d), jnp.float32),
        scratch_types=scratch,
    )
    def k(x_hbm, pos_hbm, out_hbm, *bufs):
        pos_v = bufs[0:NBUF]
        x_v = bufs[NBUF:2 * NBUF]
        in_sem = bufs[2 * NBUF:3 * NBUF]
        out_sem = bufs[3 * NBUF:4 * NBUF]

        wid = lax.axis_index("s") * 2 + lax.axis_index("c")
        i_base = wid * per_w

        def issue_in(s, p):
            i0 = i_base + s * G
            pltpu.async_copy(pos_hbm.at[pl.ds(i0, G)], pos_v[p], in_sem[p])
            pltpu.async_copy(x_hbm.at[pl.ds(i0, G)], x_v[p], in_sem[p])

        def wait_in(p):
            pltpu.make_async_copy(pos_hbm.at[pl.ds(0, G)], pos_v[p],
                                  in_sem[p]).wait()
            pltpu.make_async_copy(x_hbm.at[pl.ds(0, G)], x_v[p],
                                  in_sem[p]).wait()

        def issue_out(s, p):
            i0 = i_base + s * G
            pltpu.async_copy(x_v[p], out_hbm.at[pl.ds(i0, G)], out_sem[p])

        def wait_out(p):
            pltpu.make_async_copy(x_v[p], out_hbm.at[pl.ds(0, G)],
                                  out_sem[p]).wait()

        def compute(p):
            pv_ref = pos_v[p]
            xv_ref = x_v[p]

            def body(t, c):
                g = t >> 4
                j4 = t & 15
                for u in range(UNROLL):
                    coff = (j4 * UNROLL + u) * LANES
                    pv = pv_ref[g, pl.ds(coff, LANES)]
                    for bb in range(b):
                        xv_ref[g, bb, pl.ds(coff, LANES)] = (
                            xv_ref[g, bb, pl.ds(coff, LANES)] + pv)
                return c

            lax.fori_loop(0, G * d // (LANES * UNROLL), body, 0)

        for s in range(LEAD):
            issue_in(s, s % NBUF)

        for s in range(LEAD):
            p = s % NBUF
            wait_in(p)
            compute(p)
            issue_out(s, p)
            r = (s + LEAD) % NBUF
            if s + LEAD - NBUF >= 0:
                wait_out(r)
            issue_in(s + LEAD, r)

        n_steady = ((steps - 2 * LEAD) // NBUF) * NBUF

        def steady(it, carry):
            for p0 in range(NBUF):
                s = LEAD + it * NBUF + p0
                p = (LEAD + p0) % NBUF
                wait_in(p)
                compute(p)
                issue_out(s, p)
                r = (p + LEAD) % NBUF
                wait_out(r)
                issue_in(s + LEAD, r)
            return carry

        lax.fori_loop(0, n_steady // NBUF, steady, 0)

        for s in range(LEAD + n_steady, steps):
            p = s % NBUF
            wait_in(p)
            compute(p)
            issue_out(s, p)
            if s + LEAD < steps:
                r = (p + LEAD) % NBUF
                if s + LEAD - NBUF >= 0:
                    wait_out(r)
                issue_in(s + LEAD, r)

        for p in range(NBUF):
            wait_out(p)

    return k(x, pos_table[:chunk])


def kernel(x, pos_table):
    return _kernel_sc(x, pos_table)
